# Initial kernel scaffold; baseline (speedup 1.0000x reference)
import functools

import jax
import jax.numpy as jnp
from jax import lax
from jax.experimental import pallas as pl
from jax.experimental.pallas import tpu as pltpu
from jax.experimental.pallas import tpu_sc as plsc

NUM_USERS = 60000
NUM_ITEMS = 40000
NN = NUM_USERS + NUM_ITEMS
EMB = 32
NUM_LAYERS = 3
NW = 32
RPW = NN // NW
BLK = 128


def _layer_body(table, col, row, val, starts, out,
                starts_v, idx_v, row_v, val_v, gbuf, acc, sem):
    c = lax.axis_index("c")
    s = lax.axis_index("s")
    wid = s * 2 + c
    base = wid * RPW

    pltpu.sync_copy(starts, starts_v)

    zeros = jnp.zeros((16,), jnp.float32)

    def zbody(i, carry):
        acc[i, pl.ds(0, 16)] = zeros
        acc[i, pl.ds(16, 16)] = zeros
        return carry

    lax.fori_loop(0, RPW, zbody, 0)

    s_w = starts_v[wid]
    e_w = starts_v[wid + 1]
    k_lo = s_w // BLK
    k_hi = (e_w + BLK - 1) // BLK

    def kbody(k, carry):
        e0 = k * BLK
        pltpu.sync_copy(col.at[pl.ds(e0, BLK)], idx_v)
        pltpu.sync_copy(row.at[pl.ds(e0, BLK)], row_v)
        pltpu.sync_copy(val.at[pl.ds(e0, BLK)], val_v)
        pltpu.async_copy(table.at[idx_v], gbuf, sem).wait()

        def ebody(j, ecarry):
            d = row_v[j] - base
            ok = (d >= 0) & (d < RPW)
            dc = jnp.clip(d, 0, RPW - 1)
            sv = jnp.where(ok, val_v[j], 0.0)
            g0 = gbuf[j, pl.ds(0, 16)] * sv
            g1 = gbuf[j, pl.ds(16, 16)] * sv
            plsc.addupdate(acc.at[dc, pl.ds(0, 16)], g0)
            plsc.addupdate(acc.at[dc, pl.ds(16, 16)], g1)
            return ecarry

        lax.fori_loop(0, BLK, ebody, 0)
        return carry

    lax.fori_loop(k_lo, k_hi, kbody, 0)

    pltpu.sync_copy(acc, out.at[pl.ds(base, RPW)])


def _propagate(table, col, row, val, starts):
    mesh = plsc.VectorSubcoreMesh(core_axis_name="c", subcore_axis_name="s")
    fn = functools.partial(
        pl.kernel,
        mesh=mesh,
        out_type=jax.ShapeDtypeStruct((NN, EMB), jnp.float32),
        scratch_types=[
            pltpu.VMEM((40,), jnp.int32),
            pltpu.VMEM((BLK,), jnp.int32),
            pltpu.VMEM((BLK,), jnp.int32),
            pltpu.VMEM((BLK,), jnp.float32),
            pltpu.VMEM((BLK, EMB), jnp.float32),
            pltpu.VMEM((RPW, EMB), jnp.float32),
            pltpu.SemaphoreType.DMA,
        ],
    )(_layer_body)
    return fn(table, col, row, val, starts)


def _mean_body(a, b, c, d, o):
    o[...] = (a[...] + b[...] + c[...] + d[...]) * 0.25


def _mean4(e0, e1, e2, e3):
    R = NN * EMB // 128
    BR = 1000
    xs = [t.reshape(R, 128) for t in (e0, e1, e2, e3)]
    spec = pl.BlockSpec((BR, 128), lambda i: (i, 0))
    out = pl.pallas_call(
        _mean_body,
        grid=(R // BR,),
        in_specs=[spec] * 4,
        out_specs=spec,
        out_shape=jax.ShapeDtypeStruct((R, 128), jnp.float32),
    )(*xs)
    return out.reshape(NN, EMB)


def kernel(user_emb, item_emb, adj_row, adj_col, adj_val):
    e0 = jnp.concatenate([user_emb, item_emb], axis=0)
    col = adj_col.astype(jnp.int32)
    row = adj_row.astype(jnp.int32)
    bounds = (jnp.arange(NW + 1, dtype=jnp.int32) * RPW).astype(adj_row.dtype)
    starts = jnp.searchsorted(adj_row, bounds, side="left").astype(jnp.int32)
    starts = jnp.concatenate([starts, jnp.zeros((7,), jnp.int32)])

    e1 = _propagate(e0, col, row, adj_val, starts)
    e2 = _propagate(e1, col, row, adj_val, starts)
    e3 = _propagate(e2, col, row, adj_val, starts)

    out = _mean4(e0, e1, e2, e3)
    return out[:NUM_USERS], out[NUM_USERS:]


# speedup vs baseline: 6.8249x; 6.8249x over previous
import functools

import jax
import jax.numpy as jnp
from jax import lax
from jax.experimental import pallas as pl
from jax.experimental.pallas import tpu as pltpu
from jax.experimental.pallas import tpu_sc as plsc

NUM_USERS = 60000
NUM_ITEMS = 40000
NN = NUM_USERS + NUM_ITEMS
EMB = 32
NUM_LAYERS = 3
NW = 32
NP = 100096
RPW = NP // NW
BLK = 128


def _layer_body(table, col, row, val, starts, out,
                starts_v, idx_v, row_v, val_v, gbuf, acc, sem):
    c = lax.axis_index("c")
    s = lax.axis_index("s")
    wid = s * 2 + c
    base = wid * RPW

    pltpu.sync_copy(starts, starts_v)

    zeros = jnp.zeros((16,), jnp.float32)

    def zbody(i, carry):
        acc[i, pl.ds(0, 16)] = zeros
        acc[i, pl.ds(16, 16)] = zeros
        return carry

    lax.fori_loop(0, RPW, zbody, 0)

    s_w = starts_v[pl.ds(wid, 16)][0]
    e_w = starts_v[pl.ds(wid + 1, 16)][0]
    k_lo = s_w // BLK
    k_hi = (e_w + BLK - 1) // BLK

    def kbody(k, carry):
        e0 = k * BLK
        pltpu.sync_copy(col.at[pl.ds(e0, BLK)], idx_v)
        pltpu.sync_copy(row.at[pl.ds(e0, BLK)], row_v)
        pltpu.sync_copy(val.at[pl.ds(e0, BLK)], val_v)
        pltpu.async_copy(table.at[idx_v], gbuf, sem).wait()

        def ebody(jc, ecarry):
            j0 = jc * 16
            rv = row_v[pl.ds(j0, 16)]
            vv = val_v[pl.ds(j0, 16)]
            dv = rv - base
            okv = (dv >= 0) & (dv < RPW)
            dcv = jnp.clip(dv, 0, RPW - 1)
            svv = jnp.where(okv, vv, 0.0)
            for t in range(16):
                dc = dcv[t]
                sv = svv[t]
                g0 = gbuf[j0 + t, pl.ds(0, 16)] * sv
                g1 = gbuf[j0 + t, pl.ds(16, 16)] * sv
                plsc.addupdate(acc.at[dc, pl.ds(0, 16)], g0)
                plsc.addupdate(acc.at[dc, pl.ds(16, 16)], g1)
            return ecarry

        lax.fori_loop(0, BLK // 16, ebody, 0)
        return carry

    lax.fori_loop(k_lo, k_hi, kbody, 0)

    pltpu.sync_copy(acc, out.at[pl.ds(base, RPW)])


def _propagate(table, col, row, val, starts):
    mesh = plsc.VectorSubcoreMesh(core_axis_name="c", subcore_axis_name="s")
    fn = functools.partial(
        pl.kernel,
        mesh=mesh,
        out_type=jax.ShapeDtypeStruct((NP, EMB), jnp.float32),
        compiler_params=pltpu.CompilerParams(use_tc_tiling_on_sc=False),
        scratch_types=[
            pltpu.VMEM((48,), jnp.int32),
            pltpu.VMEM((BLK,), jnp.int32),
            pltpu.VMEM((BLK,), jnp.int32),
            pltpu.VMEM((BLK,), jnp.float32),
            pltpu.VMEM((BLK, EMB), jnp.float32),
            pltpu.VMEM((RPW, EMB), jnp.float32),
            pltpu.SemaphoreType.DMA,
        ],
    )(_layer_body)
    return fn(table, col, row, val, starts)


def _mean_body(a, b, c, d, o):
    o[...] = (a[...] + b[...] + c[...] + d[...]) * 0.25


def _mean4(e0, e1, e2, e3):
    R = NP * EMB // 128
    BR = 3128
    xs = [t.reshape(R, 128) for t in (e0, e1, e2, e3)]
    spec = pl.BlockSpec((BR, 128), lambda i: (i, 0))
    out = pl.pallas_call(
        _mean_body,
        grid=(R // BR,),
        in_specs=[spec] * 4,
        out_specs=spec,
        out_shape=jax.ShapeDtypeStruct((R, 128), jnp.float32),
    )(*xs)
    return out.reshape(NP, EMB)


def kernel(user_emb, item_emb, adj_row, adj_col, adj_val):
    e0 = jnp.concatenate(
        [user_emb, item_emb, jnp.zeros((NP - NN, EMB), jnp.float32)], axis=0)
    col = adj_col.astype(jnp.int32)
    row = adj_row.astype(jnp.int32)
    bounds = (jnp.arange(NW + 1, dtype=jnp.int32) * RPW).astype(adj_row.dtype)
    starts = jnp.searchsorted(adj_row, bounds, side="left").astype(jnp.int32)
    starts = jnp.concatenate([starts, jnp.zeros((15,), jnp.int32)])

    e1 = _propagate(e0, col, row, adj_val, starts)
    e2 = _propagate(e1, col, row, adj_val, starts)
    e3 = _propagate(e2, col, row, adj_val, starts)

    out = _mean4(e0, e1, e2, e3)
    return out[:NUM_USERS], out[NUM_USERS:NN]


# speedup vs baseline: 16.2684x; 2.3837x over previous
import functools

import jax
import jax.numpy as jnp
from jax import lax
from jax.experimental import pallas as pl
from jax.experimental.pallas import tpu as pltpu
from jax.experimental.pallas import tpu_sc as plsc

NUM_USERS = 60000
NUM_ITEMS = 40000
NN = NUM_USERS + NUM_ITEMS
EMB = 32
NUM_LAYERS = 3
NW = 32
NP = 100096
RPW = NP // NW
BLK = 128
SB = 256
EPAD = 1024


def _layer_body(table, col, row, val, starts, out,
                starts_v,
                colv0, colv1, rowv0, rowv1, valv0, valv1,
                gbuf0, gbuf1, acc,
                lsem0, lsem1, gsem0, gsem1):
    c = lax.axis_index("c")
    s = lax.axis_index("s")
    wid = s * 2 + c
    base = wid * RPW

    colv = (colv0, colv1)
    rowv = (rowv0, rowv1)
    valv = (valv0, valv1)
    gbuf = (gbuf0, gbuf1)
    lsem = (lsem0, lsem1)
    gsem = (gsem0, gsem1)

    pltpu.sync_copy(starts, starts_v)

    zeros = jnp.zeros((16,), jnp.float32)

    def zbody(i, carry):
        acc[i, pl.ds(0, 16)] = zeros
        acc[i, pl.ds(16, 16)] = zeros
        return carry

    lax.fori_loop(0, RPW, zbody, 0)

    s_w = starts_v[pl.ds(wid, 16)][0]
    e_w = starts_v[pl.ds(wid + 1, 16)][0]
    k_lo = s_w // SB
    g_cnt = (e_w + SB - 1) // SB - k_lo
    gp = (g_cnt + 1) // 2

    def fire_linear(g, b):
        e0 = (k_lo + g) * SB
        pltpu.async_copy(col.at[pl.ds(e0, SB)], colv[b], lsem[b])
        pltpu.async_copy(row.at[pl.ds(e0, SB)], rowv[b], lsem[b])
        pltpu.async_copy(val.at[pl.ds(e0, SB)], valv[b], lsem[b])

    def wait_linear(b):
        pltpu.make_async_copy(col.at[pl.ds(0, SB)], colv[b], lsem[b]).wait()
        pltpu.make_async_copy(row.at[pl.ds(0, SB)], rowv[b], lsem[b]).wait()
        pltpu.make_async_copy(val.at[pl.ds(0, SB)], valv[b], lsem[b]).wait()

    def fire_gather(b):
        for h in range(SB // BLK):
            pltpu.async_copy(table.at[colv[b].at[pl.ds(h * BLK, BLK)]],
                             gbuf[b].at[pl.ds(h * BLK, BLK)], gsem[b])

    def wait_gather(b):
        for h in range(SB // BLK):
            pltpu.make_async_copy(
                table.at[colv[b].at[pl.ds(h * BLK, BLK)]],
                gbuf[b].at[pl.ds(h * BLK, BLK)], gsem[b]).wait()

    def compute(b):
        def cbody(jc, carry):
            j0 = jc * 16
            rv = rowv[b][pl.ds(j0, 16)]
            vv = valv[b][pl.ds(j0, 16)]
            dv = rv - base
            okv = (dv >= 0) & (dv < RPW)
            dcv = jnp.clip(dv, 0, RPW - 1)
            svv = jnp.where(okv, vv, 0.0)
            for t in range(16):
                dc = dcv[t]
                sv = svv[t]
                g0 = gbuf[b][j0 + t, pl.ds(0, 16)] * sv
                g1 = gbuf[b][j0 + t, pl.ds(16, 16)] * sv
                plsc.addupdate(acc.at[dc, pl.ds(0, 16)], g0)
                plsc.addupdate(acc.at[dc, pl.ds(16, 16)], g1)
            return carry

        lax.fori_loop(0, SB // 16, cbody, 0)

    fire_linear(0, 0)
    fire_linear(1, 1)
    wait_linear(0)
    fire_gather(0)

    def body(g, b):
        wait_linear(1 - b)
        fire_gather(1 - b)
        wait_gather(b)
        compute(b)
        fire_linear(g + 2, b)

    def pair(p, carry):
        body(2 * p, 0)
        body(2 * p + 1, 1)
        return carry

    lax.fori_loop(0, gp, pair, 0)

    wait_gather(0)
    wait_linear(1)

    pltpu.sync_copy(acc, out.at[pl.ds(base, RPW)])


def _propagate(table, col, row, val, starts):
    mesh = plsc.VectorSubcoreMesh(core_axis_name="c", subcore_axis_name="s")
    fn = functools.partial(
        pl.kernel,
        mesh=mesh,
        out_type=jax.ShapeDtypeStruct((NP, EMB), jnp.float32),
        compiler_params=pltpu.CompilerParams(use_tc_tiling_on_sc=False),
        scratch_types=[
            pltpu.VMEM((48,), jnp.int32),
            pltpu.VMEM((SB,), jnp.int32),
            pltpu.VMEM((SB,), jnp.int32),
            pltpu.VMEM((SB,), jnp.int32),
            pltpu.VMEM((SB,), jnp.int32),
            pltpu.VMEM((SB,), jnp.float32),
            pltpu.VMEM((SB,), jnp.float32),
            pltpu.VMEM((SB, EMB), jnp.float32),
            pltpu.VMEM((SB, EMB), jnp.float32),
            pltpu.VMEM((RPW, EMB), jnp.float32),
            pltpu.SemaphoreType.DMA,
            pltpu.SemaphoreType.DMA,
            pltpu.SemaphoreType.DMA,
            pltpu.SemaphoreType.DMA,
        ],
    )(_layer_body)
    return fn(table, col, row, val, starts)


def _mean_body(a, b, c, d, o):
    o[...] = (a[...] + b[...] + c[...] + d[...]) * 0.25


def _mean4(e0, e1, e2, e3):
    R = NP * EMB // 128
    BR = 3128
    xs = [t.reshape(R, 128) for t in (e0, e1, e2, e3)]
    spec = pl.BlockSpec((BR, 128), lambda i: (i, 0))
    out = pl.pallas_call(
        _mean_body,
        grid=(R // BR,),
        in_specs=[spec] * 4,
        out_specs=spec,
        out_shape=jax.ShapeDtypeStruct((R, 128), jnp.float32),
    )(*xs)
    return out.reshape(NP, EMB)


def kernel(user_emb, item_emb, adj_row, adj_col, adj_val):
    e0 = jnp.concatenate(
        [user_emb, item_emb, jnp.zeros((NP - NN, EMB), jnp.float32)], axis=0)
    zpad_i = jnp.zeros((EPAD,), jnp.int32)
    col = jnp.concatenate([adj_col.astype(jnp.int32), zpad_i])
    row = jnp.concatenate([adj_row.astype(jnp.int32), zpad_i])
    vpad = jnp.concatenate([adj_val, jnp.zeros((EPAD,), jnp.float32)])
    bounds = (jnp.arange(NW + 1, dtype=jnp.int32) * RPW).astype(adj_row.dtype)
    starts = jnp.searchsorted(adj_row, bounds, side="left").astype(jnp.int32)
    starts = jnp.concatenate([starts, jnp.zeros((15,), jnp.int32)])

    e1 = _propagate(e0, col, row, vpad, starts)
    e2 = _propagate(e1, col, row, vpad, starts)
    e3 = _propagate(e2, col, row, vpad, starts)

    out = _mean4(e0, e1, e2, e3)
    return out[:NUM_USERS], out[NUM_USERS:NN]


# speedup vs baseline: 23.5432x; 1.4472x over previous
import functools

import jax
import jax.numpy as jnp
from jax import lax
from jax.experimental import pallas as pl
from jax.experimental.pallas import tpu as pltpu
from jax.experimental.pallas import tpu_sc as plsc

NUM_USERS = 60000
NUM_ITEMS = 40000
NN = NUM_USERS + NUM_ITEMS
EMB = 32
NUM_LAYERS = 3
NW = 32
NP = 100096
RPW = NP // NW
BLK = 128
SB = 256
EPAD = 1024


def _layer_body(table, col, row, val, starts, out,
                starts_v,
                colv0, colv1, rowv0, rowv1, valv0, valv1,
                gbuf0, gbuf1, acc,
                lsem0, lsem1, gsem0, gsem1):
    c = lax.axis_index("c")
    s = lax.axis_index("s")
    wid = s * 2 + c
    base = wid * RPW

    colv = (colv0, colv1)
    rowv = (rowv0, rowv1)
    valv = (valv0, valv1)
    gbuf = (gbuf0, gbuf1)
    lsem = (lsem0, lsem1)
    gsem = (gsem0, gsem1)

    pltpu.sync_copy(starts, starts_v)

    zeros = jnp.zeros((16,), jnp.float32)

    def zbody(i, carry):
        acc[i, pl.ds(0, 16)] = zeros
        acc[i, pl.ds(16, 16)] = zeros
        return carry

    lax.fori_loop(0, RPW, zbody, 0)

    s_w = starts_v[pl.ds(wid, 16)][0]
    e_w = starts_v[pl.ds(wid + 1, 16)][0]
    k_lo = s_w // SB
    g_cnt = (e_w + SB - 1) // SB - k_lo
    gp = (g_cnt + 1) // 2

    def fire_linear(g, b):
        e0 = (k_lo + g) * SB
        pltpu.async_copy(col.at[pl.ds(e0, SB)], colv[b], lsem[b])
        pltpu.async_copy(row.at[pl.ds(e0, SB)], rowv[b], lsem[b])
        pltpu.async_copy(val.at[pl.ds(e0, SB)], valv[b], lsem[b])

    def wait_linear(b):
        pltpu.make_async_copy(col.at[pl.ds(0, SB)], colv[b], lsem[b]).wait()
        pltpu.make_async_copy(row.at[pl.ds(0, SB)], rowv[b], lsem[b]).wait()
        pltpu.make_async_copy(val.at[pl.ds(0, SB)], valv[b], lsem[b]).wait()

    def fire_gather(b):
        for h in range(SB // BLK):
            pltpu.async_copy(table.at[colv[b].at[pl.ds(h * BLK, BLK)]],
                             gbuf[b].at[pl.ds(h * BLK, BLK)], gsem[b])

    def wait_gather(b):
        for h in range(SB // BLK):
            pltpu.make_async_copy(
                table.at[colv[b].at[pl.ds(h * BLK, BLK)]],
                gbuf[b].at[pl.ds(h * BLK, BLK)], gsem[b]).wait()

    def compute(b, e0):
        def chunk(jc, masked):
            j0 = jc * 16
            rv = rowv[b][pl.ds(j0, 16)]
            vv = valv[b][pl.ds(j0, 16)]
            dv = rv - base
            if masked:
                okv = (dv >= 0) & (dv < RPW)
                dv = jnp.clip(dv, 0, RPW - 1)
                vv = jnp.where(okv, vv, 0.0)
            for t in range(16):
                dc = dv[t]
                sv = vv[t]
                g0 = gbuf[b][j0 + t, pl.ds(0, 16)] * sv
                g1 = gbuf[b][j0 + t, pl.ds(16, 16)] * sv
                plsc.addupdate(acc.at[dc, pl.ds(0, 16)], g0)
                plsc.addupdate(acc.at[dc, pl.ds(16, 16)], g1)

        boundary = (e0 < s_w) | (e0 + SB > e_w)

        @pl.when(boundary)
        def _():
            @plsc.parallel_loop(0, SB // 16, 1)
            def _(jc):
                chunk(jc, True)

        @pl.when(jnp.logical_not(boundary))
        def _():
            @plsc.parallel_loop(0, SB // 16, 1)
            def _(jc):
                chunk(jc, False)

    fire_linear(0, 0)
    fire_linear(1, 1)
    wait_linear(0)
    fire_gather(0)

    def body(g, b):
        wait_linear(1 - b)
        fire_gather(1 - b)
        wait_gather(b)
        compute(b, (k_lo + g) * SB)
        fire_linear(g + 2, b)

    def pair(p, carry):
        body(2 * p, 0)
        body(2 * p + 1, 1)
        return carry

    lax.fori_loop(0, gp, pair, 0)

    wait_gather(0)
    wait_linear(1)

    pltpu.sync_copy(acc, out.at[pl.ds(base, RPW)])


def _propagate(table, col, row, val, starts):
    mesh = plsc.VectorSubcoreMesh(core_axis_name="c", subcore_axis_name="s")
    fn = functools.partial(
        pl.kernel,
        mesh=mesh,
        out_type=jax.ShapeDtypeStruct((NP, EMB), jnp.float32),
        compiler_params=pltpu.CompilerParams(use_tc_tiling_on_sc=False),
        scratch_types=[
            pltpu.VMEM((48,), jnp.int32),
            pltpu.VMEM((SB,), jnp.int32),
            pltpu.VMEM((SB,), jnp.int32),
            pltpu.VMEM((SB,), jnp.int32),
            pltpu.VMEM((SB,), jnp.int32),
            pltpu.VMEM((SB,), jnp.float32),
            pltpu.VMEM((SB,), jnp.float32),
            pltpu.VMEM((SB, EMB), jnp.float32),
            pltpu.VMEM((SB, EMB), jnp.float32),
            pltpu.VMEM((RPW, EMB), jnp.float32),
            pltpu.SemaphoreType.DMA,
            pltpu.SemaphoreType.DMA,
            pltpu.SemaphoreType.DMA,
            pltpu.SemaphoreType.DMA,
        ],
    )(_layer_body)
    return fn(table, col, row, val, starts)


def _mean_body(a, b, c, d, o):
    o[...] = (a[...] + b[...] + c[...] + d[...]) * 0.25


def _mean4(e0, e1, e2, e3):
    R = NP * EMB // 128
    BR = 3128
    xs = [t.reshape(R, 128) for t in (e0, e1, e2, e3)]
    spec = pl.BlockSpec((BR, 128), lambda i: (i, 0))
    out = pl.pallas_call(
        _mean_body,
        grid=(R // BR,),
        in_specs=[spec] * 4,
        out_specs=spec,
        out_shape=jax.ShapeDtypeStruct((R, 128), jnp.float32),
    )(*xs)
    return out.reshape(NP, EMB)


def kernel(user_emb, item_emb, adj_row, adj_col, adj_val):
    e0 = jnp.concatenate(
        [user_emb, item_emb, jnp.zeros((NP - NN, EMB), jnp.float32)], axis=0)
    zpad_i = jnp.zeros((EPAD,), jnp.int32)
    col = jnp.concatenate([adj_col.astype(jnp.int32), zpad_i])
    row = jnp.concatenate([adj_row.astype(jnp.int32), zpad_i])
    vpad = jnp.concatenate([adj_val, jnp.zeros((EPAD,), jnp.float32)])
    bounds = (jnp.arange(NW + 1, dtype=jnp.int32) * RPW).astype(adj_row.dtype)
    starts = jnp.searchsorted(adj_row, bounds, side="left").astype(jnp.int32)
    starts = jnp.concatenate([starts, jnp.zeros((15,), jnp.int32)])

    e1 = _propagate(e0, col, row, vpad, starts)
    e2 = _propagate(e1, col, row, vpad, starts)
    e3 = _propagate(e2, col, row, vpad, starts)

    out = _mean4(e0, e1, e2, e3)
    return out[:NUM_USERS], out[NUM_USERS:NN]


# speedup vs baseline: 27.8920x; 1.1847x over previous
import functools

import jax
import jax.numpy as jnp
from jax import lax
from jax.experimental import pallas as pl
from jax.experimental.pallas import tpu as pltpu
from jax.experimental.pallas import tpu_sc as plsc

NUM_USERS = 60000
NUM_ITEMS = 40000
NN = NUM_USERS + NUM_ITEMS
EMB = 32
NUM_LAYERS = 3
NW = 32
NP = 100096
RPW = NP // NW
BLK = 128
SB = 256
EPAD = 1536


def _layer_body(table, col, row, val, starts, out,
                starts_v,
                colv0, colv1, colv2, rowv0, rowv1, rowv2,
                valv0, valv1, valv2,
                gbuf0, gbuf1, gbuf2, dbuf0, dbuf1, dbuf2, shacc,
                lsem0, lsem1, lsem2, gsem0, gsem1, gsem2,
                ssem0, ssem1, ssem2, zsem):
    c = lax.axis_index("c")
    s = lax.axis_index("s")
    wid = s * 2 + c
    base = wid * RPW
    sbase = s * RPW

    colv = (colv0, colv1, colv2)
    rowv = (rowv0, rowv1, rowv2)
    valv = (valv0, valv1, valv2)
    gbuf = (gbuf0, gbuf1, gbuf2)
    dbuf = (dbuf0, dbuf1, dbuf2)
    lsem = (lsem0, lsem1, lsem2)
    gsem = (gsem0, gsem1, gsem2)
    ssem = (ssem0, ssem1, ssem2)

    pltpu.sync_copy(starts, starts_v)

    zeros = jnp.zeros((16,), jnp.float32)

    def zbody(i, carry):
        gbuf0[i, pl.ds(0, 16)] = zeros
        gbuf0[i, pl.ds(16, 16)] = zeros
        return carry

    lax.fori_loop(0, SB, zbody, 0)

    for q in range(RPW // SB):
        pltpu.async_copy(gbuf0, shacc.at[pl.ds(sbase + q * SB, SB)], zsem)
    pltpu.async_copy(gbuf0.at[pl.ds(0, RPW % SB)],
                     shacc.at[pl.ds(sbase + (RPW // SB) * SB, RPW % SB)], zsem)
    for q in range(RPW // SB):
        pltpu.make_async_copy(gbuf0, shacc.at[pl.ds(0, SB)], zsem).wait()
    pltpu.make_async_copy(gbuf0.at[pl.ds(0, RPW % SB)],
                          shacc.at[pl.ds(0, RPW % SB)], zsem).wait()

    s_w = starts_v[pl.ds(wid, 16)][0]
    e_w = starts_v[pl.ds(wid + 1, 16)][0]
    k_lo = s_w // SB
    g_cnt = (e_w + SB - 1) // SB - k_lo
    gp = (g_cnt + 2) // 3

    def fire_linear(g, b):
        e0 = (k_lo + g) * SB
        pltpu.async_copy(col.at[pl.ds(e0, SB)], colv[b], lsem[b])
        pltpu.async_copy(row.at[pl.ds(e0, SB)], rowv[b], lsem[b])
        pltpu.async_copy(val.at[pl.ds(e0, SB)], valv[b], lsem[b])

    def wait_linear(b):
        pltpu.make_async_copy(col.at[pl.ds(0, SB)], colv[b], lsem[b]).wait()
        pltpu.make_async_copy(row.at[pl.ds(0, SB)], rowv[b], lsem[b]).wait()
        pltpu.make_async_copy(val.at[pl.ds(0, SB)], valv[b], lsem[b]).wait()

    def fire_gather(b):
        for h in range(SB // BLK):
            pltpu.async_copy(table.at[colv[b].at[pl.ds(h * BLK, BLK)]],
                             gbuf[b].at[pl.ds(h * BLK, BLK)], gsem[b])

    def wait_gather(b):
        for h in range(SB // BLK):
            pltpu.make_async_copy(
                table.at[colv[b].at[pl.ds(h * BLK, BLK)]],
                gbuf[b].at[pl.ds(h * BLK, BLK)], gsem[b]).wait()

    def compute(b, e0):
        def chunk(jc, masked):
            j0 = jc * 16
            rv = rowv[b][pl.ds(j0, 16)]
            vv = valv[b][pl.ds(j0, 16)]
            dv = rv - base
            if masked:
                okv = (dv >= 0) & (dv < RPW)
                dv = jnp.clip(dv, 0, RPW - 1)
                vv = jnp.where(okv, vv, 0.0)
            h = jc // (BLK // 16)
            p = (jc % (BLK // 16)) * 16
            dbuf[b][h, pl.ds(p, 16)] = dv + sbase
            for t in range(16):
                sv = vv[t]
                gbuf[b][j0 + t, pl.ds(0, 16)] = gbuf[b][j0 + t, pl.ds(0, 16)] * sv
                gbuf[b][j0 + t, pl.ds(16, 16)] = gbuf[b][j0 + t, pl.ds(16, 16)] * sv

        boundary = (e0 < s_w) | (e0 + SB > e_w)

        @pl.when(boundary)
        def _():
            @plsc.parallel_loop(0, SB // 16, 1)
            def _(jc):
                chunk(jc, True)

        @pl.when(jnp.logical_not(boundary))
        def _():
            @plsc.parallel_loop(0, SB // 16, 1)
            def _(jc):
                chunk(jc, False)

    def fire_scatter(b):
        for h in range(SB // BLK):
            pltpu.async_copy(gbuf[b].at[pl.ds(h * BLK, BLK)],
                             shacc.at[dbuf[b].at[h]], ssem[b], add=True)

    def wait_scatter(b):
        for h in range(SB // BLK):
            pltpu.make_async_copy(gbuf[b].at[pl.ds(h * BLK, BLK)],
                                  shacc.at[dbuf[b].at[h]], ssem[b]).wait()

    fire_linear(0, 0)
    fire_linear(1, 1)
    wait_linear(0)
    fire_gather(0)

    def body(g, b):
        nb = (b + 1) % 3
        wait_linear(nb)

        @pl.when(g >= 2)
        def _():
            wait_scatter(nb)

        fire_gather(nb)
        wait_gather(b)
        compute(b, (k_lo + g) * SB)
        fire_scatter(b)
        fire_linear(g + 2, (b + 2) % 3)

    def triple(p, carry):
        body(3 * p, 0)
        body(3 * p + 1, 1)
        body(3 * p + 2, 2)
        return carry

    lax.fori_loop(0, gp, triple, 0)

    wait_gather(0)
    wait_linear(1)

    @pl.when(gp > 0)
    def _():
        wait_scatter(1)
        wait_scatter(2)

    pltpu.sync_copy(shacc.at[pl.ds(sbase, RPW)], out.at[pl.ds(base, RPW)])


def _propagate(table, col, row, val, starts):
    mesh = plsc.VectorSubcoreMesh(core_axis_name="c", subcore_axis_name="s")
    fn = functools.partial(
        pl.kernel,
        mesh=mesh,
        out_type=jax.ShapeDtypeStruct((NP, EMB), jnp.float32),
        compiler_params=pltpu.CompilerParams(use_tc_tiling_on_sc=False),
        scratch_types=[
            pltpu.VMEM((48,), jnp.int32),
            pltpu.VMEM((SB,), jnp.int32),
            pltpu.VMEM((SB,), jnp.int32),
            pltpu.VMEM((SB,), jnp.int32),
            pltpu.VMEM((SB,), jnp.int32),
            pltpu.VMEM((SB,), jnp.int32),
            pltpu.VMEM((SB,), jnp.int32),
            pltpu.VMEM((SB,), jnp.float32),
            pltpu.VMEM((SB,), jnp.float32),
            pltpu.VMEM((SB,), jnp.float32),
            pltpu.VMEM((SB, EMB), jnp.float32),
            pltpu.VMEM((SB, EMB), jnp.float32),
            pltpu.VMEM((SB, EMB), jnp.float32),
            pltpu.VMEM((SB // BLK, BLK), jnp.int32),
            pltpu.VMEM((SB // BLK, BLK), jnp.int32),
            pltpu.VMEM((SB // BLK, BLK), jnp.int32),
            pltpu.VMEM_SHARED((16 * RPW, EMB), jnp.float32),
            pltpu.SemaphoreType.DMA,
            pltpu.SemaphoreType.DMA,
            pltpu.SemaphoreType.DMA,
            pltpu.SemaphoreType.DMA,
            pltpu.SemaphoreType.DMA,
            pltpu.SemaphoreType.DMA,
            pltpu.SemaphoreType.DMA,
            pltpu.SemaphoreType.DMA,
            pltpu.SemaphoreType.DMA,
            pltpu.SemaphoreType.DMA,
        ],
    )(_layer_body)
    return fn(table, col, row, val, starts)


def _mean_body(a, b, c, d, o):
    o[...] = (a[...] + b[...] + c[...] + d[...]) * 0.25


def _mean4(e0, e1, e2, e3):
    R = NP * EMB // 128
    BR = 3128
    xs = [t.reshape(R, 128) for t in (e0, e1, e2, e3)]
    spec = pl.BlockSpec((BR, 128), lambda i: (i, 0))
    out = pl.pallas_call(
        _mean_body,
        grid=(R // BR,),
        in_specs=[spec] * 4,
        out_specs=spec,
        out_shape=jax.ShapeDtypeStruct((R, 128), jnp.float32),
    )(*xs)
    return out.reshape(NP, EMB)


def kernel(user_emb, item_emb, adj_row, adj_col, adj_val):
    e0 = jnp.concatenate(
        [user_emb, item_emb, jnp.zeros((NP - NN, EMB), jnp.float32)], axis=0)
    zpad_i = jnp.zeros((EPAD,), jnp.int32)
    col = jnp.concatenate([adj_col.astype(jnp.int32), zpad_i])
    row = jnp.concatenate([adj_row.astype(jnp.int32), zpad_i])
    vpad = jnp.concatenate([adj_val, jnp.zeros((EPAD,), jnp.float32)])
    bounds = (jnp.arange(NW + 1, dtype=jnp.int32) * RPW).astype(adj_row.dtype)
    starts = jnp.searchsorted(adj_row, bounds, side="left").astype(jnp.int32)
    starts = jnp.concatenate([starts, jnp.zeros((15,), jnp.int32)])

    e1 = _propagate(e0, col, row, vpad, starts)
    e2 = _propagate(e1, col, row, vpad, starts)
    e3 = _propagate(e2, col, row, vpad, starts)

    out = _mean4(e0, e1, e2, e3)
    return out[:NUM_USERS], out[NUM_USERS:NN]


# speedup vs baseline: 27.9293x; 1.0013x over previous
import functools

import jax
import jax.numpy as jnp
from jax import lax
from jax.experimental import pallas as pl
from jax.experimental.pallas import tpu as pltpu
from jax.experimental.pallas import tpu_sc as plsc

NUM_USERS = 60000
NUM_ITEMS = 40000
NN = NUM_USERS + NUM_ITEMS
EMB = 32
NUM_LAYERS = 3
NW = 32
NP = 100096
RPW = NP // NW
NE = 1600000
BLK = 128
SB = 256


def _make_layer_body(final):
    def body(*refs):
        if final:
            (table, col, row, val, starts, e0t, e1t, out,
             starts_v,
             colv0, colv1, colv2, rowv0, rowv1, rowv2,
             valv0, valv1, valv2,
             gbuf0, gbuf1, gbuf2, dbuf0, dbuf1, dbuf2, shacc,
             lsem0, lsem1, lsem2, gsem0, gsem1, gsem2,
             ssem0, ssem1, ssem2, zsem) = refs
        else:
            (table, col, row, val, starts, out,
             starts_v,
             colv0, colv1, colv2, rowv0, rowv1, rowv2,
             valv0, valv1, valv2,
             gbuf0, gbuf1, gbuf2, dbuf0, dbuf1, dbuf2, shacc,
             lsem0, lsem1, lsem2, gsem0, gsem1, gsem2,
             ssem0, ssem1, ssem2, zsem) = refs

        c = lax.axis_index("c")
        s = lax.axis_index("s")
        wid = s * 2 + c
        base = wid * RPW
        sbase = s * RPW

        colv = (colv0, colv1, colv2)
        rowv = (rowv0, rowv1, rowv2)
        valv = (valv0, valv1, valv2)
        gbuf = (gbuf0, gbuf1, gbuf2)
        dbuf = (dbuf0, dbuf1, dbuf2)
        lsem = (lsem0, lsem1, lsem2)
        gsem = (gsem0, gsem1, gsem2)
        ssem = (ssem0, ssem1, ssem2)

        pltpu.sync_copy(starts, starts_v)

        zeros = jnp.zeros((16,), jnp.float32)

        def zbody(i, carry):
            gbuf0[i, pl.ds(0, 16)] = zeros
            gbuf0[i, pl.ds(16, 16)] = zeros
            return carry

        lax.fori_loop(0, SB, zbody, 0)

        for q in range(RPW // SB):
            pltpu.async_copy(gbuf0, shacc.at[pl.ds(sbase + q * SB, SB)], zsem)
        pltpu.async_copy(gbuf0.at[pl.ds(0, RPW % SB)],
                         shacc.at[pl.ds(sbase + (RPW // SB) * SB, RPW % SB)],
                         zsem)
        for q in range(RPW // SB):
            pltpu.make_async_copy(gbuf0, shacc.at[pl.ds(0, SB)], zsem).wait()
        pltpu.make_async_copy(gbuf0.at[pl.ds(0, RPW % SB)],
                              shacc.at[pl.ds(0, RPW % SB)], zsem).wait()

        s_w = starts_v[pl.ds(wid, 16)][0]
        e_w = starts_v[pl.ds(wid + 1, 16)][0]
        k_lo = s_w // SB
        g_cnt = (e_w + SB - 1) // SB - k_lo
        gp = (g_cnt + 2) // 3

        def fire_linear(g, b):
            e0 = jnp.minimum((k_lo + g) * SB, NE - SB)
            pltpu.async_copy(col.at[pl.ds(e0, SB)], colv[b], lsem[b])
            pltpu.async_copy(row.at[pl.ds(e0, SB)], rowv[b], lsem[b])
            pltpu.async_copy(val.at[pl.ds(e0, SB)], valv[b], lsem[b])

        def wait_linear(b):
            pltpu.make_async_copy(col.at[pl.ds(0, SB)], colv[b], lsem[b]).wait()
            pltpu.make_async_copy(row.at[pl.ds(0, SB)], rowv[b], lsem[b]).wait()
            pltpu.make_async_copy(val.at[pl.ds(0, SB)], valv[b], lsem[b]).wait()

        def fire_gather(b):
            for h in range(SB // BLK):
                pltpu.async_copy(table.at[colv[b].at[pl.ds(h * BLK, BLK)]],
                                 gbuf[b].at[pl.ds(h * BLK, BLK)], gsem[b])

        def wait_gather(b):
            for h in range(SB // BLK):
                pltpu.make_async_copy(
                    table.at[colv[b].at[pl.ds(h * BLK, BLK)]],
                    gbuf[b].at[pl.ds(h * BLK, BLK)], gsem[b]).wait()

        def compute(b, e0):
            def chunk(jc, masked):
                j0 = jc * 16
                rv = rowv[b][pl.ds(j0, 16)]
                vv = valv[b][pl.ds(j0, 16)]
                dv = rv - base
                if masked:
                    okv = (dv >= 0) & (dv < RPW)
                    okv = okv & (e0 + j0 + lax.iota(jnp.int32, 16) < e_w)
                    dv = jnp.clip(dv, 0, RPW - 1)
                    vv = jnp.where(okv, vv, 0.0)
                h = jc // (BLK // 16)
                p = (jc % (BLK // 16)) * 16
                dbuf[b][h, pl.ds(p, 16)] = dv + sbase
                for t in range(16):
                    sv = vv[t]
                    gbuf[b][j0 + t, pl.ds(0, 16)] = (
                        gbuf[b][j0 + t, pl.ds(0, 16)] * sv)
                    gbuf[b][j0 + t, pl.ds(16, 16)] = (
                        gbuf[b][j0 + t, pl.ds(16, 16)] * sv)

            boundary = (e0 < s_w) | (e0 + SB > e_w)

            @pl.when(boundary)
            def _():
                @plsc.parallel_loop(0, SB // 16, 1)
                def _(jc):
                    chunk(jc, True)

            @pl.when(jnp.logical_not(boundary))
            def _():
                @plsc.parallel_loop(0, SB // 16, 1)
                def _(jc):
                    chunk(jc, False)

        def fire_scatter(b):
            for h in range(SB // BLK):
                pltpu.async_copy(gbuf[b].at[pl.ds(h * BLK, BLK)],
                                 shacc.at[dbuf[b].at[h]], ssem[b], add=True)

        def wait_scatter(b):
            for h in range(SB // BLK):
                pltpu.make_async_copy(gbuf[b].at[pl.ds(h * BLK, BLK)],
                                      shacc.at[dbuf[b].at[h]], ssem[b]).wait()

        fire_linear(0, 0)
        fire_linear(1, 1)
        wait_linear(0)
        fire_gather(0)

        def pipe_body(g, b):
            nb = (b + 1) % 3
            wait_linear(nb)

            @pl.when(g >= 2)
            def _():
                wait_scatter(nb)

            fire_gather(nb)
            wait_gather(b)
            compute(b, (k_lo + g) * SB)
            fire_scatter(b)
            fire_linear(g + 2, (b + 2) % 3)

        def triple(p, carry):
            pipe_body(3 * p, 0)
            pipe_body(3 * p + 1, 1)
            pipe_body(3 * p + 2, 2)
            return carry

        lax.fori_loop(0, gp, triple, 0)

        wait_gather(0)
        wait_linear(1)

        @pl.when(gp > 0)
        def _():
            wait_scatter(1)
            wait_scatter(2)

        if not final:
            pltpu.sync_copy(shacc.at[pl.ds(sbase, RPW)],
                            out.at[pl.ds(base, RPW)])
        else:
            def mean_rows(r, rows):
                pltpu.async_copy(e0t.at[pl.ds(base + r, rows)],
                                 gbuf0.at[pl.ds(0, rows)], lsem0)
                pltpu.async_copy(e1t.at[pl.ds(base + r, rows)],
                                 gbuf0.at[pl.ds(BLK, rows)], lsem1)
                pltpu.async_copy(table.at[pl.ds(base + r, rows)],
                                 gbuf1.at[pl.ds(0, rows)], lsem2)
                pltpu.make_async_copy(e0t.at[pl.ds(base + r, rows)],
                                      gbuf0.at[pl.ds(0, rows)], lsem0).wait()
                pltpu.make_async_copy(e1t.at[pl.ds(base + r, rows)],
                                      gbuf0.at[pl.ds(BLK, rows)], lsem1).wait()
                pltpu.make_async_copy(table.at[pl.ds(base + r, rows)],
                                      gbuf1.at[pl.ds(0, rows)], lsem2).wait()
                pltpu.sync_copy(shacc.at[pl.ds(sbase + r, rows)],
                                gbuf1.at[pl.ds(BLK, rows)])

                @plsc.parallel_loop(0, rows, 1)
                def _(i):
                    for hh in (0, 16):
                        m = (gbuf0[i, pl.ds(hh, 16)]
                             + gbuf0[BLK + i, pl.ds(hh, 16)]
                             + gbuf1[i, pl.ds(hh, 16)]
                             + gbuf1[BLK + i, pl.ds(hh, 16)]) * 0.25
                        gbuf2[i, pl.ds(hh, 16)] = m

                pltpu.sync_copy(gbuf2.at[pl.ds(0, rows)],
                                out.at[pl.ds(base + r, rows)])

            def mean_loop(q, carry):
                mean_rows(q * BLK, BLK)
                return carry

            lax.fori_loop(0, RPW // BLK, mean_loop, 0)
            mean_rows((RPW // BLK) * BLK, RPW % BLK)

    return body


_SCRATCH = [
    pltpu.VMEM((48,), jnp.int32),
    pltpu.VMEM((SB,), jnp.int32),
    pltpu.VMEM((SB,), jnp.int32),
    pltpu.VMEM((SB,), jnp.int32),
    pltpu.VMEM((SB,), jnp.int32),
    pltpu.VMEM((SB,), jnp.int32),
    pltpu.VMEM((SB,), jnp.int32),
    pltpu.VMEM((SB,), jnp.float32),
    pltpu.VMEM((SB,), jnp.float32),
    pltpu.VMEM((SB,), jnp.float32),
    pltpu.VMEM((SB, EMB), jnp.float32),
    pltpu.VMEM((SB, EMB), jnp.float32),
    pltpu.VMEM((SB, EMB), jnp.float32),
    pltpu.VMEM((SB // BLK, BLK), jnp.int32),
    pltpu.VMEM((SB // BLK, BLK), jnp.int32),
    pltpu.VMEM((SB // BLK, BLK), jnp.int32),
    pltpu.VMEM_SHARED((16 * RPW, EMB), jnp.float32),
    pltpu.SemaphoreType.DMA,
    pltpu.SemaphoreType.DMA,
    pltpu.SemaphoreType.DMA,
    pltpu.SemaphoreType.DMA,
    pltpu.SemaphoreType.DMA,
    pltpu.SemaphoreType.DMA,
    pltpu.SemaphoreType.DMA,
    pltpu.SemaphoreType.DMA,
    pltpu.SemaphoreType.DMA,
    pltpu.SemaphoreType.DMA,
]


def _propagate(table, col, row, val, starts):
    mesh = plsc.VectorSubcoreMesh(core_axis_name="c", subcore_axis_name="s")
    fn = functools.partial(
        pl.kernel,
        mesh=mesh,
        out_type=jax.ShapeDtypeStruct((NP, EMB), jnp.float32),
        compiler_params=pltpu.CompilerParams(use_tc_tiling_on_sc=False),
        scratch_types=_SCRATCH,
    )(_make_layer_body(False))
    return fn(table, col, row, val, starts)


def _propagate_mean(table, col, row, val, starts, e0t, e1t):
    mesh = plsc.VectorSubcoreMesh(core_axis_name="c", subcore_axis_name="s")
    fn = functools.partial(
        pl.kernel,
        mesh=mesh,
        out_type=jax.ShapeDtypeStruct((NP, EMB), jnp.float32),
        compiler_params=pltpu.CompilerParams(use_tc_tiling_on_sc=False),
        scratch_types=_SCRATCH,
    )(_make_layer_body(True))
    return fn(table, col, row, val, starts, e0t, e1t)


def kernel(user_emb, item_emb, adj_row, adj_col, adj_val):
    e0 = jnp.concatenate(
        [user_emb, item_emb, jnp.zeros((NP - NN, EMB), jnp.float32)], axis=0)
    col = adj_col.astype(jnp.int32)
    row = adj_row.astype(jnp.int32)
    bounds = (jnp.arange(NW + 1, dtype=jnp.int32) * RPW).astype(adj_row.dtype)
    starts = jnp.searchsorted(adj_row, bounds, side="left").astype(jnp.int32)
    starts = jnp.concatenate([starts, jnp.zeros((15,), jnp.int32)])

    e1 = _propagate(e0, col, row, adj_val, starts)
    e2 = _propagate(e1, col, row, adj_val, starts)
    out = _propagate_mean(e2, col, row, adj_val, starts, e0, e1)

    return out[:NUM_USERS], out[NUM_USERS:NN]
